# baseline (speedup 1.0000x reference)
import functools

import jax
import jax.numpy as jnp
from jax import lax
from jax.experimental import pallas as pl
from jax.experimental.pallas import tpu as pltpu, tpu_sc as plsc

B = 1024
V = 100000
NC = 2
NS = 16
L = 16
NW = NC * NS
NB = 8
NG = NW // NB
VS = 63200
SW = VS // NG
VCH = 200
NCHK = SW // VCH
CHT = 1600
NTC = (V - VS) // CHT
BL = B // NB
NEG = float("-inf")

_mesh = plsc.VectorSubcoreMesh(
    core_axis_name="c", subcore_axis_name="s", num_cores=NC, num_subcores=NS
)


@functools.partial(
    pl.kernel,
    out_type=(
        jax.ShapeDtypeStruct((NG * B,), jnp.float32),
        jax.ShapeDtypeStruct((NG * B,), jnp.float32),
    ),
    mesh=_mesh,
    scratch_types=[
        pltpu.VMEM((VCH, BL), jnp.float32),
        pltpu.VMEM((VCH, BL), jnp.float32),
        pltpu.VMEM((BL,), jnp.float32),
        pltpu.VMEM((BL,), jnp.float32),
        pltpu.SemaphoreType.DMA,
        pltpu.SemaphoreType.DMA,
    ],
)
def _topk_kernel(xt_hbm, m1_hbm, m2_hbm, bufa, bufb, m1_v, m2_v, sa, sb):
    wid = lax.axis_index("s") * NC + lax.axis_index("c")
    jb = wid % NB
    gs = wid // NB
    v_lo = gs * SW
    b_lo = jb * BL

    def start(k, buf, sem):
        pltpu.make_async_copy(
            xt_hbm.at[
                pl.ds(pl.multiple_of(v_lo + k * VCH, 8), VCH),
                pl.ds(pl.multiple_of(b_lo, 128), BL),
            ],
            buf, sem,
        ).start()

    def wait(buf, sem):
        pltpu.make_async_copy(
            xt_hbm.at[pl.ds(0, VCH), pl.ds(0, BL)], buf, sem
        ).wait()

    start(0, bufa, sa)
    start(1, bufb, sb)

    def do_chunk(k, buf, sem, m1s, m2s):
        wait(buf, sem)

        def hot(i, carry):
            a = list(carry[:8])
            b = list(carry[8:])
            for u in range(8):
                x = buf[i, pl.ds(u * L, L)]
                b[u] = jnp.maximum(b[u], jnp.minimum(a[u], x))
                a[u] = jnp.maximum(a[u], x)
            return tuple(a) + tuple(b)

        out = lax.fori_loop(0, VCH, hot, tuple(m1s) + tuple(m2s), unroll=4)

        @pl.when(k + 2 < NCHK)
        def _():
            start(k + 2, buf, sem)

        return list(out[:8]), list(out[8:])

    neg = jnp.full((L,), NEG, jnp.float32)
    init = tuple(neg for _ in range(16))

    def pair(m, carry):
        m1s, m2s = list(carry[:8]), list(carry[8:])
        m1s, m2s = do_chunk(2 * m, bufa, sa, m1s, m2s)
        m1s, m2s = do_chunk(2 * m + 1, bufb, sb, m1s, m2s)
        return tuple(m1s) + tuple(m2s)

    fin = lax.fori_loop(0, NCHK // 2, pair, init)
    m1s, m2s = list(fin[:8]), list(fin[8:])
    m1s, m2s = do_chunk(NCHK - 1, bufa, sa, m1s, m2s)

    for u in range(8):
        m1_v[pl.ds(u * L, L)] = m1s[u]
        m2_v[pl.ds(u * L, L)] = m2s[u]
    pltpu.sync_copy(m1_v, m1_hbm.at[pl.ds(gs * B + b_lo, BL)])
    pltpu.sync_copy(m2_v, m2_hbm.at[pl.ds(gs * B + b_lo, BL)])


def _tc_body(x_ref, m1_ref, m2_ref):
    @pl.when(pl.program_id(0) == 0)
    def _():
        m1_ref[...] = jnp.full((B,), NEG, jnp.float32)
        m2_ref[...] = jnp.full((B,), NEG, jnp.float32)

    def row(i, carry):
        a, b = carry
        x = x_ref[i, :]
        b = jnp.maximum(b, jnp.minimum(a, x))
        a = jnp.maximum(a, x)
        return a, b

    a, b = lax.fori_loop(
        0, CHT, row, (m1_ref[...], m2_ref[...]), unroll=8
    )
    m1_ref[...] = a
    m2_ref[...] = b


_tc_topk = pl.pallas_call(
    _tc_body,
    grid=(NTC,),
    in_specs=[pl.BlockSpec((CHT, B), lambda i: (VS // CHT + i, 0))],
    out_specs=(
        pl.BlockSpec((B,), lambda i: (0,)),
        pl.BlockSpec((B,), lambda i: (0,)),
    ),
    out_shape=(
        jax.ShapeDtypeStruct((B,), jnp.float32),
        jax.ShapeDtypeStruct((B,), jnp.float32),
    ),
)


def _comb_body(m1_ref, m2_ref, m1t_ref, m2t_ref, c_ref, o_ref):
    M1 = m1_ref[pl.ds(0, B)]
    M2 = m2_ref[pl.ds(0, B)]
    for gs in range(1, NG):
        B1 = m1_ref[pl.ds(gs * B, B)]
        B2 = m2_ref[pl.ds(gs * B, B)]
        M2 = jnp.maximum(jnp.minimum(M1, B1), jnp.maximum(M2, B2))
        M1 = jnp.maximum(M1, B1)
    B1 = m1t_ref[...]
    B2 = m2t_ref[...]
    M2 = jnp.maximum(jnp.minimum(M1, B1), jnp.maximum(M2, B2))
    M1 = jnp.maximum(M1, B1)
    c = c_ref[...]
    o_ref[...] = c - jnp.where(c == M1, M2, M1)


_tc_combine = pl.pallas_call(
    _comb_body,
    out_shape=jax.ShapeDtypeStruct((B,), jnp.float32),
)


def kernel(inputs, targets):
    tg = targets.astype(jnp.int32)
    xt = inputs.T
    cls = jnp.take_along_axis(inputs, tg[:, None], axis=1)[:, 0]
    m1, m2 = _topk_kernel(xt)
    m1t, m2t = _tc_topk(xt)
    return _tc_combine(m1, m2, m1t, m2t, cls)


# speedup vs baseline: 1.0201x; 1.0201x over previous
import functools

import jax
import jax.numpy as jnp
from jax import lax
from jax.experimental import pallas as pl
from jax.experimental.pallas import tpu as pltpu, tpu_sc as plsc

B = 1024
V = 100000
NC = 2
NS = 16
L = 16
NW = NC * NS
NB = 8
NG = NW // NB
VS = 61600
SW = VS // NG
VCH = 280
NCHK = SW // VCH
CHT = 1200
NTC = (V - VS) // CHT
BL = B // NB
NEG = float("-inf")

_mesh = plsc.VectorSubcoreMesh(
    core_axis_name="c", subcore_axis_name="s", num_cores=NC, num_subcores=NS
)


@functools.partial(
    pl.kernel,
    out_type=(
        jax.ShapeDtypeStruct((NG * B,), jnp.float32),
        jax.ShapeDtypeStruct((NG * B,), jnp.float32),
    ),
    mesh=_mesh,
    scratch_types=[
        pltpu.VMEM((VCH, BL), jnp.float32),
        pltpu.VMEM((VCH, BL), jnp.float32),
        pltpu.VMEM((BL,), jnp.float32),
        pltpu.VMEM((BL,), jnp.float32),
        pltpu.SemaphoreType.DMA,
        pltpu.SemaphoreType.DMA,
    ],
)
def _topk_kernel(xt_hbm, m1_hbm, m2_hbm, bufa, bufb, m1_v, m2_v, sa, sb):
    wid = lax.axis_index("s") * NC + lax.axis_index("c")
    jb = wid % NB
    gs = wid // NB
    v_lo = gs * SW
    b_lo = jb * BL

    def start(k, buf, sem):
        pltpu.make_async_copy(
            xt_hbm.at[
                pl.ds(pl.multiple_of(v_lo + k * VCH, 8), VCH),
                pl.ds(pl.multiple_of(b_lo, 128), BL),
            ],
            buf, sem,
        ).start()

    def wait(buf, sem):
        pltpu.make_async_copy(
            xt_hbm.at[pl.ds(0, VCH), pl.ds(0, BL)], buf, sem
        ).wait()

    start(0, bufa, sa)
    start(1, bufb, sb)

    def do_chunk(k, buf, sem, m1s, m2s):
        wait(buf, sem)

        def hot(i, carry):
            a = list(carry[:8])
            b = list(carry[8:])
            for u in range(8):
                x = buf[i, pl.ds(u * L, L)]
                b[u] = jnp.maximum(b[u], jnp.minimum(a[u], x))
                a[u] = jnp.maximum(a[u], x)
            return tuple(a) + tuple(b)

        out = lax.fori_loop(0, VCH, hot, tuple(m1s) + tuple(m2s), unroll=4)

        @pl.when(k + 2 < NCHK)
        def _():
            start(k + 2, buf, sem)

        return list(out[:8]), list(out[8:])

    neg = jnp.full((L,), NEG, jnp.float32)
    init = tuple(neg for _ in range(16))

    def pair(m, carry):
        m1s, m2s = list(carry[:8]), list(carry[8:])
        m1s, m2s = do_chunk(2 * m, bufa, sa, m1s, m2s)
        m1s, m2s = do_chunk(2 * m + 1, bufb, sb, m1s, m2s)
        return tuple(m1s) + tuple(m2s)

    fin = lax.fori_loop(0, NCHK // 2, pair, init)
    m1s, m2s = list(fin[:8]), list(fin[8:])
    m1s, m2s = do_chunk(NCHK - 1, bufa, sa, m1s, m2s)

    for u in range(8):
        m1_v[pl.ds(u * L, L)] = m1s[u]
        m2_v[pl.ds(u * L, L)] = m2s[u]
    pltpu.sync_copy(m1_v, m1_hbm.at[pl.ds(gs * B + b_lo, BL)])
    pltpu.sync_copy(m2_v, m2_hbm.at[pl.ds(gs * B + b_lo, BL)])


def _tc_body(x_ref, m1_ref, m2_ref):
    @pl.when(pl.program_id(0) == 0)
    def _():
        m1_ref[...] = jnp.full((B,), NEG, jnp.float32)
        m2_ref[...] = jnp.full((B,), NEG, jnp.float32)

    def row(i, carry):
        a, b = carry
        x = x_ref[i, :]
        b = jnp.maximum(b, jnp.minimum(a, x))
        a = jnp.maximum(a, x)
        return a, b

    a, b = lax.fori_loop(
        0, CHT, row, (m1_ref[...], m2_ref[...]), unroll=8
    )
    m1_ref[...] = a
    m2_ref[...] = b


_tc_topk = pl.pallas_call(
    _tc_body,
    grid=(NTC,),
    in_specs=[pl.BlockSpec((CHT, B), lambda i: (VS // CHT + i, 0))],
    out_specs=(
        pl.BlockSpec((B,), lambda i: (0,)),
        pl.BlockSpec((B,), lambda i: (0,)),
    ),
    out_shape=(
        jax.ShapeDtypeStruct((B,), jnp.float32),
        jax.ShapeDtypeStruct((B,), jnp.float32),
    ),
)


def _comb_body(m1_ref, m2_ref, m1t_ref, m2t_ref, c_ref, o_ref):
    M1 = m1_ref[pl.ds(0, B)]
    M2 = m2_ref[pl.ds(0, B)]
    for gs in range(1, NG):
        B1 = m1_ref[pl.ds(gs * B, B)]
        B2 = m2_ref[pl.ds(gs * B, B)]
        M2 = jnp.maximum(jnp.minimum(M1, B1), jnp.maximum(M2, B2))
        M1 = jnp.maximum(M1, B1)
    B1 = m1t_ref[...]
    B2 = m2t_ref[...]
    M2 = jnp.maximum(jnp.minimum(M1, B1), jnp.maximum(M2, B2))
    M1 = jnp.maximum(M1, B1)
    c = c_ref[...]
    o_ref[...] = c - jnp.where(c == M1, M2, M1)


_tc_combine = pl.pallas_call(
    _comb_body,
    out_shape=jax.ShapeDtypeStruct((B,), jnp.float32),
)


def kernel(inputs, targets):
    tg = targets.astype(jnp.int32)
    xt = inputs.T
    cls = jnp.take_along_axis(inputs, tg[:, None], axis=1)[:, 0]
    m1, m2 = _topk_kernel(xt)
    m1t, m2t = _tc_topk(xt)
    return _tc_combine(m1, m2, m1t, m2t, cls)
